# Initial kernel scaffold; baseline (speedup 1.0000x reference)
import jax
import jax.numpy as jnp
from jax.experimental import pallas as pl


def kernel(x, edge_attr, edge_index, batch, params):
    raise NotImplementedError("write your pallas kernel here")


# speedup vs baseline: 1.0479x; 1.0479x over previous
import functools
import math

import jax
import jax.numpy as jnp
import numpy as np
from jax.experimental import pallas as pl

_C = 8
_F = 72
_BN = 2000
_BE = 8000
_EPS_NA = 1e-5


def _expand_o3(W0, W1, W2, m0, m1, m2):
    F_in = m0 + 3 * m1 + 5 * m2
    out = jnp.zeros((F_in, _F), dtype=jnp.float32)
    out = out.at[:m0, :_C].set(W0 / np.sqrt(m0))
    out = out.at[m0:m0 + 3 * m1, _C:_C + 3 * _C].set(
        jnp.kron(W1, jnp.eye(3, dtype=jnp.float32)) / np.sqrt(m1))
    out = out.at[m0 + 3 * m1:, _C + 3 * _C:].set(
        jnp.kron(W2, jnp.eye(5, dtype=jnp.float32)) / np.sqrt(m2))
    return out


def _expand_wd(Wd0, Wd1, Wd2):
    out = jnp.zeros((_F, _F), dtype=jnp.float32)
    out = out.at[:_C, :_C].set(Wd0)
    out = out.at[_C:_C + 24, _C:_C + 24].set(
        jnp.kron(Wd1, jnp.eye(3, dtype=jnp.float32)))
    out = out.at[_C + 24:, _C + 24:].set(
        jnp.kron(Wd2, jnp.eye(5, dtype=jnp.float32)))
    return out


def _group_matrix():
    g = np.zeros((_F, _F), dtype=np.float32)
    g[:_C, :_C] = np.eye(_C)
    g[_C:_C + 24, _C:_C + 24] = np.kron(np.eye(_C), np.ones((3, 3)))
    g[_C + 24:, _C + 24:] = np.kron(np.eye(_C), np.ones((5, 5)))
    return jnp.asarray(g)


def _full_spec(shape):
    return pl.BlockSpec(shape, lambda i: tuple(0 for _ in shape))


def _f0_body(x_ref, ea_ref, f_ref):
    r = ea_ref[...]
    nrm = jnp.sqrt(jnp.sum(r * r, axis=1, keepdims=True))
    n = r / (nrm + 1e-9)
    xx, yy, zz = n[:, 0:1], n[:, 1:2], n[:, 2:3]
    s3 = float(np.sqrt(3.0)); s15 = float(np.sqrt(15.0)); s5 = float(np.sqrt(5.0))
    sh = jnp.concatenate([
        jnp.ones_like(xx), s3 * xx, s3 * yy, s3 * zz,
        s15 * xx * yy, s15 * yy * zz, 0.5 * s5 * (3.0 * zz * zz - 1.0),
        s15 * xx * zz, 0.5 * s15 * (xx * xx - yy * yy)], axis=1)
    f_ref[...] = jnp.concatenate([x_ref[...], sh], axis=1)


def _qkv_body(f_ref, wq_ref, wk_ref, wv_ref, wd_ref, g_ref, qt_ref, k_ref, v_ref):
    f = f_ref[...]
    G = g_ref[...]

    def na(w):
        lin = jnp.dot(f, w, preferred_element_type=jnp.float32)
        nrm = jnp.sqrt(jnp.dot(lin * lin, G, preferred_element_type=jnp.float32))
        return lin * (nrm / (nrm + _EPS_NA))

    q = na(wq_ref[...])
    k = na(wk_ref[...])
    v = na(wv_ref[...])
    scale = 1.0 / float(np.sqrt(3.0 * _C * _C))
    qt_ref[...] = jnp.dot(q, wd_ref[...], preferred_element_type=jnp.float32) * scale
    k_ref[...] = k
    v_ref[...] = v


def _edge_dot_body(qe_ref, ke_ref, ex_ref):
    d = jnp.sum(qe_ref[...] * ke_ref[...], axis=1, keepdims=True)
    ex_ref[...] = jnp.exp(d)


def _edge_msg_body(ex_ref, z_ref, ve_ref, msg_ref):
    alpha = ex_ref[...] / z_ref[...]
    msg_ref[...] = jnp.sqrt(jnp.maximum(alpha, 0.0)) * ve_ref[...]


def _readout_body(f_ref, wol_ref, w1_ref, b1_ref, w2_ref, b2_ref, out_ref):
    xs = jnp.dot(f_ref[...][:, :_C], wol_ref[...],
                 preferred_element_type=jnp.float32) * (1.0 / float(np.sqrt(_C)))
    nrm = jnp.sqrt(jnp.sum(xs * xs, axis=1, keepdims=True))
    xs = xs / (nrm + 1e-12)
    xs = jnp.maximum(xs, 0.0)
    xs = jnp.maximum(jnp.dot(xs, w1_ref[...], preferred_element_type=jnp.float32)
                     + b1_ref[...], 0.0)
    xs = jnp.maximum(jnp.dot(xs, w2_ref[...], preferred_element_type=jnp.float32)
                     + b2_ref[...], 0.0)
    out_ref[...] = xs


def _final_body(sums_ref, cnt_ref, wout_ref, bout_ref, out_ref):
    m = sums_ref[...] / jnp.maximum(cnt_ref[...], 1.0)
    l = jnp.dot(m, wout_ref[...], preferred_element_type=jnp.float32) + bout_ref[...]
    l = l - jnp.max(l, axis=1, keepdims=True)
    e = jnp.exp(l)
    out_ref[...] = e / jnp.sum(e, axis=1, keepdims=True)


def _node_qkv(f, wq, wk, wv, wd, G):
    n, fin = f.shape
    grid = (n // _BN,)
    specs_w = [_full_spec(wq.shape), _full_spec(wk.shape), _full_spec(wv.shape),
               _full_spec(wd.shape), _full_spec(G.shape)]
    return pl.pallas_call(
        _qkv_body,
        grid=grid,
        in_specs=[pl.BlockSpec((_BN, fin), lambda i: (i, 0))] + specs_w,
        out_specs=[pl.BlockSpec((_BN, _F), lambda i: (i, 0))] * 3,
        out_shape=[jax.ShapeDtypeStruct((n, _F), jnp.float32)] * 3,
    )(f, wq, wk, wv, wd, G)


def _edge_dot(qe, ke):
    e = qe.shape[0]
    return pl.pallas_call(
        _edge_dot_body,
        grid=(e // _BE,),
        in_specs=[pl.BlockSpec((_BE, _F), lambda i: (i, 0))] * 2,
        out_specs=pl.BlockSpec((_BE, 1), lambda i: (i, 0)),
        out_shape=jax.ShapeDtypeStruct((e, 1), jnp.float32),
    )(qe, ke)


def _edge_msg(ex, zd, ve):
    e = ve.shape[0]
    return pl.pallas_call(
        _edge_msg_body,
        grid=(e // _BE,),
        in_specs=[pl.BlockSpec((_BE, 1), lambda i: (i, 0)),
                  pl.BlockSpec((_BE, 1), lambda i: (i, 0)),
                  pl.BlockSpec((_BE, _F), lambda i: (i, 0))],
        out_specs=pl.BlockSpec((_BE, _F), lambda i: (i, 0)),
        out_shape=jax.ShapeDtypeStruct((e, _F), jnp.float32),
    )(ex, zd, ve)


def _layer(f, edge_src, edge_dst, n, wq, wk, wv, wd, G):
    qt, k, v = _node_qkv(f, wq, wk, wv, wd, G)
    qe = qt[edge_dst]
    ke = k[edge_src]
    ve = v[edge_src]
    ex = _edge_dot(qe, ke)
    z = jax.ops.segment_sum(ex[:, 0], edge_dst, num_segments=n)
    z = jnp.where(z == 0.0, 1.0, z)
    msg = _edge_msg(ex, z[edge_dst][:, None], ve)
    return jax.ops.segment_sum(msg, edge_dst, num_segments=n)


def kernel(x, edge_attr, edge_index, batch, params):
    n = x.shape[0]
    edge_dst = edge_index[0]
    edge_src = edge_index[1]
    G = _group_matrix()

    f = pl.pallas_call(
        _f0_body,
        grid=(n // _BN,),
        in_specs=[pl.BlockSpec((_BN, 9), lambda i: (i, 0)),
                  pl.BlockSpec((_BN, 3), lambda i: (i, 0))],
        out_specs=pl.BlockSpec((_BN, 18), lambda i: (i, 0)),
        out_shape=jax.ShapeDtypeStruct((n, 18), jnp.float32),
    )(x, edge_attr)

    for li, p in enumerate(params['layers']):
        m0, m1, m2 = (10, 1, 1) if li == 0 else (_C, _C, _C)
        wq = _expand_o3(p['Wq0'], p['Wq1'], p['Wq2'], m0, m1, m2)
        wk = _expand_o3(p['Wk0'], p['Wk1'], p['Wk2'], m0, m1, m2)
        wv = _expand_o3(p['Wv0'], p['Wv1'], p['Wv2'], m0, m1, m2)
        wd = _expand_wd(p['Wd0'], p['Wd1'], p['Wd2'])
        upd = _layer(f, edge_src, edge_dst, n, wq, wk, wv, wd, G)
        f = upd if li == 0 else upd + f

    H = params['Wol'].shape[1]
    xs = pl.pallas_call(
        _readout_body,
        grid=(n // _BN,),
        in_specs=[pl.BlockSpec((_BN, _F), lambda i: (i, 0)),
                  _full_spec(params['Wol'].shape),
                  _full_spec(params['Wlin'][0].shape),
                  _full_spec((1, H)),
                  _full_spec(params['Wlin'][1].shape),
                  _full_spec((1, H))],
        out_specs=pl.BlockSpec((_BN, H), lambda i: (i, 0)),
        out_shape=jax.ShapeDtypeStruct((n, H), jnp.float32),
    )(f, params['Wol'], params['Wlin'][0], params['blin'][0][None, :],
      params['Wlin'][1], params['blin'][1][None, :])

    gi = batch[edge_src]
    feats = xs[edge_src]
    b = params['bout'].shape[0]
    nb = params['Wout'].shape[0]
    n_groups = 64
    sums = jax.ops.segment_sum(feats, gi, num_segments=n_groups)
    cnt = jax.ops.segment_sum(jnp.ones_like(gi, dtype=jnp.float32), gi,
                              num_segments=n_groups)
    out = pl.pallas_call(
        _final_body,
        grid=(1,),
        in_specs=[_full_spec((n_groups, H)),
                  _full_spec((n_groups, 1)),
                  _full_spec((nb, b)),
                  _full_spec((1, b))],
        out_specs=_full_spec((n_groups, b)),
        out_shape=jax.ShapeDtypeStruct((n_groups, b), jnp.float32),
    )(sums, cnt[:, None], params['Wout'], params['bout'][None, :])
    return out


# speedup vs baseline: 1.3251x; 1.2646x over previous
import functools
import math

import jax
import jax.numpy as jnp
import numpy as np
from jax.experimental import pallas as pl

_C = 8
_F = 72
_BN = 2000
_BE = 8000
_EPS_NA = 1e-5


def _expand_o3(W0, W1, W2, m0, m1, m2):
    F_in = m0 + 3 * m1 + 5 * m2
    out = jnp.zeros((F_in, _F), dtype=jnp.float32)
    out = out.at[:m0, :_C].set(W0 / np.sqrt(m0))
    out = out.at[m0:m0 + 3 * m1, _C:_C + 3 * _C].set(
        jnp.kron(W1, jnp.eye(3, dtype=jnp.float32)) / np.sqrt(m1))
    out = out.at[m0 + 3 * m1:, _C + 3 * _C:].set(
        jnp.kron(W2, jnp.eye(5, dtype=jnp.float32)) / np.sqrt(m2))
    return out


def _expand_wd(Wd0, Wd1, Wd2):
    out = jnp.zeros((_F, _F), dtype=jnp.float32)
    out = out.at[:_C, :_C].set(Wd0)
    out = out.at[_C:_C + 24, _C:_C + 24].set(
        jnp.kron(Wd1, jnp.eye(3, dtype=jnp.float32)))
    out = out.at[_C + 24:, _C + 24:].set(
        jnp.kron(Wd2, jnp.eye(5, dtype=jnp.float32)))
    return out


def _group_matrix():
    g = np.zeros((_F, _F), dtype=np.float32)
    g[:_C, :_C] = np.eye(_C)
    g[_C:_C + 24, _C:_C + 24] = np.kron(np.eye(_C), np.ones((3, 3)))
    g[_C + 24:, _C + 24:] = np.kron(np.eye(_C), np.ones((5, 5)))
    return jnp.asarray(g)


def _full_spec(shape):
    return pl.BlockSpec(shape, lambda i: tuple(0 for _ in shape))


def _f0_body(x_ref, ea_ref, f_ref):
    r = ea_ref[...]
    nrm = jnp.sqrt(jnp.sum(r * r, axis=1, keepdims=True))
    n = r / (nrm + 1e-9)
    xx, yy, zz = n[:, 0:1], n[:, 1:2], n[:, 2:3]
    s3 = float(np.sqrt(3.0)); s15 = float(np.sqrt(15.0)); s5 = float(np.sqrt(5.0))
    sh = jnp.concatenate([
        jnp.ones_like(xx), s3 * xx, s3 * yy, s3 * zz,
        s15 * xx * yy, s15 * yy * zz, 0.5 * s5 * (3.0 * zz * zz - 1.0),
        s15 * xx * zz, 0.5 * s15 * (xx * xx - yy * yy)], axis=1)
    f_ref[...] = jnp.concatenate([x_ref[...], sh], axis=1)


def _qkv_body(f_ref, wq_ref, wk_ref, wv_ref, wd_ref, g_ref, qt_ref, k_ref, v_ref):
    f = f_ref[...]
    G = g_ref[...]

    def na(w):
        lin = jnp.dot(f, w, preferred_element_type=jnp.float32)
        nrm = jnp.sqrt(jnp.dot(lin * lin, G, preferred_element_type=jnp.float32))
        return lin * (nrm / (nrm + _EPS_NA))

    q = na(wq_ref[...])
    k = na(wk_ref[...])
    v = na(wv_ref[...])
    scale = 1.0 / float(np.sqrt(3.0 * _C * _C))
    qt_ref[...] = jnp.dot(q, wd_ref[...], preferred_element_type=jnp.float32) * scale
    k_ref[...] = k
    v_ref[...] = v


def _edge_dot_body(qe_ref, ke_ref, ex_ref):
    d = jnp.sum(qe_ref[...] * ke_ref[...], axis=1, keepdims=True)
    ex_ref[...] = jnp.exp(d)


def _edge_msg_body(ex_ref, z_ref, ve_ref, msg_ref):
    alpha = ex_ref[...] / z_ref[...]
    msg_ref[...] = jnp.sqrt(jnp.maximum(alpha, 0.0)) * ve_ref[...]


def _readout_body(f_ref, deg_ref, wol_ref, w1_ref, b1_ref, w2_ref, b2_ref, out_ref):
    xs = jnp.dot(f_ref[...][:, :_C], wol_ref[...],
                 preferred_element_type=jnp.float32) * (1.0 / float(np.sqrt(_C)))
    nrm = jnp.sqrt(jnp.sum(xs * xs, axis=1, keepdims=True))
    xs = xs / (nrm + 1e-12)
    xs = jnp.maximum(xs, 0.0)
    xs = jnp.maximum(jnp.dot(xs, w1_ref[...], preferred_element_type=jnp.float32)
                     + b1_ref[...], 0.0)
    xs = jnp.maximum(jnp.dot(xs, w2_ref[...], preferred_element_type=jnp.float32)
                     + b2_ref[...], 0.0)
    out_ref[...] = xs * deg_ref[...]


def _final_body(sums_ref, cnt_ref, wout_ref, bout_ref, out_ref):
    m = sums_ref[...] / jnp.maximum(cnt_ref[...], 1.0)
    l = jnp.dot(m, wout_ref[...], preferred_element_type=jnp.float32) + bout_ref[...]
    l = l - jnp.max(l, axis=1, keepdims=True)
    e = jnp.exp(l)
    out_ref[...] = e / jnp.sum(e, axis=1, keepdims=True)


def _node_qkv(f, wq, wk, wv, wd, G):
    n, fin = f.shape
    grid = (n // _BN,)
    specs_w = [_full_spec(wq.shape), _full_spec(wk.shape), _full_spec(wv.shape),
               _full_spec(wd.shape), _full_spec(G.shape)]
    return pl.pallas_call(
        _qkv_body,
        grid=grid,
        in_specs=[pl.BlockSpec((_BN, fin), lambda i: (i, 0))] + specs_w,
        out_specs=[pl.BlockSpec((_BN, _F), lambda i: (i, 0))] * 3,
        out_shape=[jax.ShapeDtypeStruct((n, _F), jnp.float32)] * 3,
    )(f, wq, wk, wv, wd, G)


def _edge_dot(qe, ke):
    e = qe.shape[0]
    return pl.pallas_call(
        _edge_dot_body,
        grid=(e // _BE,),
        in_specs=[pl.BlockSpec((_BE, _F), lambda i: (i, 0))] * 2,
        out_specs=pl.BlockSpec((_BE, 1), lambda i: (i, 0)),
        out_shape=jax.ShapeDtypeStruct((e, 1), jnp.float32),
    )(qe, ke)


def _edge_msg(ex, zd, ve):
    e = ve.shape[0]
    return pl.pallas_call(
        _edge_msg_body,
        grid=(e // _BE,),
        in_specs=[pl.BlockSpec((_BE, 1), lambda i: (i, 0)),
                  pl.BlockSpec((_BE, 1), lambda i: (i, 0)),
                  pl.BlockSpec((_BE, _F), lambda i: (i, 0))],
        out_specs=pl.BlockSpec((_BE, _F), lambda i: (i, 0)),
        out_shape=jax.ShapeDtypeStruct((e, _F), jnp.float32),
    )(ex, zd, ve)


def _layer(f, edge_src, edge_dst, n, wq, wk, wv, wd, G):
    qt, k, v = _node_qkv(f, wq, wk, wv, wd, G)
    qe = qt[edge_dst]
    ke = k[edge_src]
    ve = v[edge_src]
    ex = _edge_dot(qe, ke)
    z = jax.ops.segment_sum(ex[:, 0], edge_dst, num_segments=n)
    z = jnp.where(z == 0.0, 1.0, z)
    msg = _edge_msg(ex, z[edge_dst][:, None], ve)
    return jax.ops.segment_sum(msg, edge_dst, num_segments=n)


def kernel(x, edge_attr, edge_index, batch, params):
    n = x.shape[0]
    edge_dst = edge_index[0]
    edge_src = edge_index[1]
    G = _group_matrix()

    f = pl.pallas_call(
        _f0_body,
        grid=(n // _BN,),
        in_specs=[pl.BlockSpec((_BN, 9), lambda i: (i, 0)),
                  pl.BlockSpec((_BN, 3), lambda i: (i, 0))],
        out_specs=pl.BlockSpec((_BN, 18), lambda i: (i, 0)),
        out_shape=jax.ShapeDtypeStruct((n, 18), jnp.float32),
    )(x, edge_attr)

    for li, p in enumerate(params['layers']):
        m0, m1, m2 = (10, 1, 1) if li == 0 else (_C, _C, _C)
        wq = _expand_o3(p['Wq0'], p['Wq1'], p['Wq2'], m0, m1, m2)
        wk = _expand_o3(p['Wk0'], p['Wk1'], p['Wk2'], m0, m1, m2)
        wv = _expand_o3(p['Wv0'], p['Wv1'], p['Wv2'], m0, m1, m2)
        wd = _expand_wd(p['Wd0'], p['Wd1'], p['Wd2'])
        upd = _layer(f, edge_src, edge_dst, n, wq, wk, wv, wd, G)
        f = upd if li == 0 else upd + f

    H = params['Wol'].shape[1]
    deg = jax.ops.segment_sum(jnp.ones((edge_src.shape[0],), jnp.float32),
                              edge_src, num_segments=n)
    xs_w = pl.pallas_call(
        _readout_body,
        grid=(n // _BN,),
        in_specs=[pl.BlockSpec((_BN, _F), lambda i: (i, 0)),
                  pl.BlockSpec((_BN, 1), lambda i: (i, 0)),
                  _full_spec(params['Wol'].shape),
                  _full_spec(params['Wlin'][0].shape),
                  _full_spec((1, H)),
                  _full_spec(params['Wlin'][1].shape),
                  _full_spec((1, H))],
        out_specs=pl.BlockSpec((_BN, H), lambda i: (i, 0)),
        out_shape=jax.ShapeDtypeStruct((n, H), jnp.float32),
    )(f, deg[:, None], params['Wol'], params['Wlin'][0], params['blin'][0][None, :],
      params['Wlin'][1], params['blin'][1][None, :])

    b = params['bout'].shape[0]
    nb = params['Wout'].shape[0]
    n_groups = 64
    sums = jax.ops.segment_sum(xs_w, batch, num_segments=n_groups,
                               indices_are_sorted=True)
    cnt = jax.ops.segment_sum(deg, batch, num_segments=n_groups,
                              indices_are_sorted=True)
    out = pl.pallas_call(
        _final_body,
        grid=(1,),
        in_specs=[_full_spec((n_groups, H)),
                  _full_spec((n_groups, 1)),
                  _full_spec((nb, b)),
                  _full_spec((1, b))],
        out_specs=_full_spec((n_groups, b)),
        out_shape=jax.ShapeDtypeStruct((n_groups, b), jnp.float32),
    )(sums, cnt[:, None], params['Wout'], params['bout'][None, :])
    return out
